# Initial kernel scaffold; baseline (speedup 1.0000x reference)
import jax
import jax.numpy as jnp
from jax.experimental import pallas as pl


def kernel(x, edge_index, W_gcn, b_gcn, W_gat, att_src, att_dst, b_gat, W_sage_l, b_sage_l, W_sage_r, W_fus, b_fus):
    raise NotImplementedError("write your pallas kernel here")


# speedup vs baseline: 19.4349x; 19.4349x over previous
import dataclasses

import jax
import jax.numpy as jnp
from jax import lax
from jax.experimental import pallas as pl
from jax.experimental.pallas import tpu as pltpu
from jax.experimental.pallas import tpu_sc as plsc

N = 10000
NP = 10240
D = 128
NC = 2
NS = 16
L = 16
NW = NC * NS
EPT = 10240
E_PAD = NW * EPT
BLK = 128
RPT = NP // NS
GB = 1024
GRID = NP // GB

_mesh = plsc.VectorSubcoreMesh(core_axis_name="c", subcore_axis_name="s")

_sc_params = pltpu.CompilerParams()
if "needs_layout_passes" in pltpu.CompilerParams.__dataclass_fields__:
    _sc_params = dataclasses.replace(_sc_params, needs_layout_passes=False)


def _pre_body(x_ref, w_ref, o_ref):
    o_ref[...] = jnp.dot(x_ref[...], w_ref[...],
                         preferred_element_type=jnp.float32)


def _tc_pre(x, wcat):
    return pl.pallas_call(
        _pre_body,
        grid=(GRID,),
        in_specs=[pl.BlockSpec((GB, D), lambda i: (i, 0)),
                  pl.BlockSpec((D, 3 * D), lambda i: (0, 0))],
        out_specs=pl.BlockSpec((GB, 3 * D), lambda i: (i, 0)),
        out_shape=jax.ShapeDtypeStruct((NP, 3 * D), jnp.float32),
    )(x, wcat)


def _mid_body(cntp_ref, xw_ref, u_ref):
    cnt = jnp.sum(cntp_ref[...], axis=0)
    dinv = lax.rsqrt(cnt + 1.0)
    u_ref[...] = dinv[:, None] * xw_ref[...]


def _tc_mid(cnt_parts, xw):
    return pl.pallas_call(
        _mid_body,
        grid=(GRID,),
        in_specs=[pl.BlockSpec((NW, GB), lambda i: (0, i)),
                  pl.BlockSpec((GB, D), lambda i: (i, 0))],
        out_specs=pl.BlockSpec((GB, D), lambda i: (i, 0)),
        out_shape=jax.ShapeDtypeStruct((NP, D), jnp.float32),
    )(cnt_parts, xw)


def _post_body(cntp_ref, asump_ref, ssage_ref, sgcn_ref, sgat_ref, x_ref,
               xwg_ref, wsl_ref, wsr_ref, wfus_ref, bg_ref, bga_ref, bsl_ref,
               bf_ref, o_ref):
    cnt = jnp.sum(cntp_ref[...], axis=0)
    asum_e = jnp.sum(asump_ref[...], axis=0)
    s_sage = ssage_ref[0] + ssage_ref[1]
    s_gcn = sgcn_ref[0] + sgcn_ref[1]
    s_gat = sgat_ref[0] + sgat_ref[1]
    xwg = xwg_ref[...]
    xw = xwg[:, 0:D]
    xg = xwg[:, D:2 * D]
    a_s = xwg[:, 2 * D:2 * D + 1]
    a_d = xwg[:, 2 * D + 1:2 * D + 2]

    dinv = lax.rsqrt(cnt + 1.0)[:, None]
    h_gcn = jnp.maximum(dinv * s_gcn + dinv * dinv * xw + bg_ref[...], 0.0)

    al = a_s + a_d
    ae_self = jnp.exp(jnp.maximum(al, 0.2 * al))
    denom = asum_e[:, None] + ae_self + 1e-16
    h_gat = jnp.maximum((s_gat + ae_self * xg) / denom + bga_ref[...], 0.0)

    mean = s_sage / jnp.maximum(cnt, 1.0)[:, None]
    h_sage = jnp.maximum(
        jnp.dot(mean, wsl_ref[...], preferred_element_type=jnp.float32)
        + bsl_ref[...]
        + jnp.dot(x_ref[...], wsr_ref[...], preferred_element_type=jnp.float32),
        0.0)

    wfus = wfus_ref[...]
    o_ref[...] = (
        jnp.dot(h_gcn, wfus[0:D], preferred_element_type=jnp.float32)
        + jnp.dot(h_gat, wfus[D:2 * D], preferred_element_type=jnp.float32)
        + jnp.dot(h_sage, wfus[2 * D:3 * D], preferred_element_type=jnp.float32)
        + bf_ref[...])


def _tc_post(cnt_parts, asum_parts, s_sage, s_gcn, s_gat, x, xwg,
             W_sage_l, W_sage_r, W_fus, b_gcn, b_gat, b_sage_l, b_fus):
    return pl.pallas_call(
        _post_body,
        grid=(GRID,),
        in_specs=[
            pl.BlockSpec((NW, GB), lambda i: (0, i)),
            pl.BlockSpec((NW, GB), lambda i: (0, i)),
            pl.BlockSpec((NC, GB, D), lambda i: (0, i, 0)),
            pl.BlockSpec((NC, GB, D), lambda i: (0, i, 0)),
            pl.BlockSpec((NC, GB, D), lambda i: (0, i, 0)),
            pl.BlockSpec((GB, D), lambda i: (i, 0)),
            pl.BlockSpec((GB, 3 * D), lambda i: (i, 0)),
            pl.BlockSpec((D, D), lambda i: (0, 0)),
            pl.BlockSpec((D, D), lambda i: (0, 0)),
            pl.BlockSpec((3 * D, D), lambda i: (0, 0)),
            pl.BlockSpec((1, D), lambda i: (0, 0)),
            pl.BlockSpec((1, D), lambda i: (0, 0)),
            pl.BlockSpec((1, D), lambda i: (0, 0)),
            pl.BlockSpec((1, D), lambda i: (0, 0)),
        ],
        out_specs=pl.BlockSpec((GB, D), lambda i: (i, 0)),
        out_shape=jax.ShapeDtypeStruct((NP, D), jnp.float32),
    )(cnt_parts, asum_parts, s_sage, s_gcn, s_gat, x, xwg,
      W_sage_l, W_sage_r, W_fus,
      b_gcn.reshape(1, D), b_gat.reshape(1, D), b_sage_l.reshape(1, D),
      b_fus.reshape(1, D))


def _sc0_body(row_hbm, col_hbm, asrc_hbm, adst_hbm,
              ae_hbm, asum_hbm, cnt_hbm,
              asrc_v, adst_v, row_v, col_v, ae_v, asum_v, cnt_v):
    c = lax.axis_index("c")
    s = lax.axis_index("s")
    wid = s * NC + c
    base = wid * EPT

    pltpu.sync_copy(asrc_hbm, asrc_v)
    pltpu.sync_copy(adst_hbm, adst_v)
    pltpu.sync_copy(row_hbm.at[pl.ds(base, EPT)], row_v)
    pltpu.sync_copy(col_hbm.at[pl.ds(base, EPT)], col_v)

    zero16 = jnp.zeros((L,), jnp.float32)

    @pl.loop(0, NP, step=L)
    def _(i):
        asum_v[pl.ds(i, L)] = zero16
        cnt_v[pl.ds(i, L)] = zero16

    ones = jnp.ones((L,), jnp.float32)

    @pl.loop(0, EPT, step=L)
    def _(i):
        r = row_v[pl.ds(i, L)]
        cc = col_v[pl.ds(i, L)]
        sa = plsc.load_gather(asrc_v, [r])
        da = plsc.load_gather(adst_v, [cc])
        al = sa + da
        ae = jnp.exp(jnp.maximum(al, 0.2 * al))
        ae_v[pl.ds(i, L)] = ae
        plsc.addupdate_scatter(asum_v, [cc], ae)
        plsc.addupdate_scatter(cnt_v, [cc], ones)

    pltpu.sync_copy(ae_v, ae_hbm.at[pl.ds(base, EPT)])
    pltpu.sync_copy(asum_v, asum_hbm.at[wid])
    pltpu.sync_copy(cnt_v, cnt_hbm.at[wid])


def _sc_edge_scalars(row, col, a_src, a_dst):
    kern = pl.kernel(
        _sc0_body,
        out_type=(jax.ShapeDtypeStruct((E_PAD,), jnp.float32),
                  jax.ShapeDtypeStruct((NW, NP), jnp.float32),
                  jax.ShapeDtypeStruct((NW, NP), jnp.float32)),
        mesh=_mesh,
        scratch_types=[
            pltpu.VMEM((NP,), jnp.float32),
            pltpu.VMEM((NP,), jnp.float32),
            pltpu.VMEM((EPT,), jnp.int32),
            pltpu.VMEM((EPT,), jnp.int32),
            pltpu.VMEM((EPT,), jnp.float32),
            pltpu.VMEM((NP,), jnp.float32),
            pltpu.VMEM((NP,), jnp.float32),
        ],
        compiler_params=_sc_params,
    )
    return kern(row, col, a_src, a_dst)


def _make_agg_body(scaled):
    def body(*refs):
        if scaled:
            (tab_hbm, row_hbm, col_hbm, ae_hbm, out_hbm,
             acc_sh, idxr_v, idxc_v, rows_v, ae_v, sem) = refs
        else:
            (tab_hbm, row_hbm, col_hbm, out_hbm,
             acc_sh, idxr_v, idxc_v, rows_v, sem) = refs
        c = lax.axis_index("c")
        s = lax.axis_index("s")
        wid = s * NC + c
        ebase = wid * EPT
        rbase = s * RPT

        zero16 = jnp.zeros((L,), jnp.float32)

        @pl.loop(0, BLK)
        def _(j):
            for k in range(D // L):
                rows_v[j, pl.ds(k * L, L)] = zero16

        @pl.loop(0, RPT, step=BLK)
        def _(r0):
            pltpu.sync_copy(rows_v, acc_sh.at[pl.ds(rbase + r0, BLK)])

        plsc.subcore_barrier()

        @pl.loop(0, EPT, step=BLK)
        def _(e0):
            pltpu.sync_copy(row_hbm.at[pl.ds(ebase + e0, BLK)], idxr_v)
            pltpu.sync_copy(col_hbm.at[pl.ds(ebase + e0, BLK)], idxc_v)
            pltpu.async_copy(tab_hbm.at[idxr_v], rows_v, sem).wait()
            if scaled:
                pltpu.sync_copy(ae_hbm.at[pl.ds(ebase + e0, BLK)], ae_v)

                @pl.loop(0, BLK)
                def _(j):
                    a = plsc.load_gather(ae_v, [jnp.full((L,), j, jnp.int32)])
                    for k in range(D // L):
                        sl = pl.ds(k * L, L)
                        rows_v[j, sl] = rows_v[j, sl] * a
            pltpu.sync_copy(rows_v, acc_sh.at[idxc_v], add=True)

        plsc.subcore_barrier()

        @pl.loop(0, RPT, step=BLK)
        def _(r0):
            pltpu.sync_copy(acc_sh.at[pl.ds(rbase + r0, BLK)], rows_v)
            pltpu.sync_copy(rows_v, out_hbm.at[c].at[pl.ds(rbase + r0, BLK)])

    return body


def _make_agg(scaled):
    scratch = [
        pltpu.VMEM_SHARED((NP, D), jnp.float32),
        pltpu.VMEM((BLK,), jnp.int32),
        pltpu.VMEM((BLK,), jnp.int32),
        pltpu.VMEM((BLK, D), jnp.float32),
    ]
    if scaled:
        scratch.append(pltpu.VMEM((BLK,), jnp.float32))
    scratch.append(pltpu.SemaphoreType.DMA)
    return pl.kernel(
        _make_agg_body(scaled),
        out_type=jax.ShapeDtypeStruct((NC, NP, D), jnp.float32),
        mesh=_mesh,
        scratch_types=scratch,
        compiler_params=_sc_params,
    )


_sc_agg_plain = _make_agg(False)
_sc_agg_scaled = _make_agg(True)


def kernel(x, edge_index, W_gcn, b_gcn, W_gat, att_src, att_dst, b_gat,
           W_sage_l, b_sage_l, W_sage_r, W_fus, b_fus):
    row = edge_index[0]
    col = edge_index[1]
    npad = E_PAD - row.shape[0]
    ar = jnp.arange(npad, dtype=jnp.int32)
    row_p = jnp.concatenate([row, (ar * 37) % N])
    col_p = jnp.concatenate([col, N + ar % (NP - N)])

    x_p = jnp.zeros((NP, D), jnp.float32).at[:N].set(x)
    att2 = (jnp.zeros((D, D), jnp.float32)
            .at[:, 0].set(W_gat @ att_src).at[:, 1].set(W_gat @ att_dst))
    wcat = jnp.concatenate([W_gcn, W_gat, att2], axis=1)

    xwg = _tc_pre(x_p, wcat)
    xw = xwg[:, 0:D]
    xg = xwg[:, D:2 * D]
    a_src = xwg[:, 2 * D]
    a_dst = xwg[:, 2 * D + 1]

    ae, asum_parts, cnt_parts = _sc_edge_scalars(row_p, col_p, a_src, a_dst)
    s_sage = _sc_agg_plain(x_p, row_p, col_p)
    u = _tc_mid(cnt_parts, xw)
    s_gcn = _sc_agg_plain(u, row_p, col_p)
    s_gat = _sc_agg_scaled(xg, row_p, col_p, ae)

    out = _tc_post(cnt_parts, asum_parts, s_sage, s_gcn, s_gat, x_p, xwg,
                   W_sage_l, W_sage_r, W_fus, b_gcn, b_gat, b_sage_l, b_fus)
    return out[:N]


# speedup vs baseline: 39.1154x; 2.0126x over previous
import dataclasses

import jax
import jax.numpy as jnp
from jax import lax
from jax.experimental import pallas as pl
from jax.experimental.pallas import tpu as pltpu
from jax.experimental.pallas import tpu_sc as plsc

N = 10000
NP = 10240
D = 128
NC = 2
NS = 16
L = 16
NW = NC * NS
EPT = 10240
E_PAD = NW * EPT
BLK = 128
RPT = NP // NS
GB = 1024
GRID = NP // GB

_mesh = plsc.VectorSubcoreMesh(core_axis_name="c", subcore_axis_name="s")

_sc_params = pltpu.CompilerParams()
if "needs_layout_passes" in pltpu.CompilerParams.__dataclass_fields__:
    _sc_params = dataclasses.replace(_sc_params, needs_layout_passes=False)


def _pre_body(x_ref, w_ref, o_ref):
    o_ref[...] = jnp.dot(x_ref[...], w_ref[...],
                         preferred_element_type=jnp.float32)


def _tc_pre(x, wcat):
    return pl.pallas_call(
        _pre_body,
        grid=(GRID,),
        in_specs=[pl.BlockSpec((GB, D), lambda i: (i, 0)),
                  pl.BlockSpec((D, 3 * D), lambda i: (0, 0))],
        out_specs=pl.BlockSpec((GB, 3 * D), lambda i: (i, 0)),
        out_shape=jax.ShapeDtypeStruct((NP, 3 * D), jnp.float32),
    )(x, wcat)


def _mid_body(cntp_ref, xw_ref, u_ref):
    cnt = jnp.sum(cntp_ref[...], axis=0)
    dinv = lax.rsqrt(cnt + 1.0)
    u_ref[...] = dinv[:, None] * xw_ref[...]


def _tc_mid(cnt_parts, xw):
    return pl.pallas_call(
        _mid_body,
        grid=(GRID,),
        in_specs=[pl.BlockSpec((NW, GB), lambda i: (0, i)),
                  pl.BlockSpec((GB, D), lambda i: (i, 0))],
        out_specs=pl.BlockSpec((GB, D), lambda i: (i, 0)),
        out_shape=jax.ShapeDtypeStruct((NP, D), jnp.float32),
    )(cnt_parts, xw)


def _post_body(cntp_ref, asump_ref, ssage_ref, sgcn_ref, sgat_ref, x_ref,
               xwg_ref, wsl_ref, wsr_ref, wfus_ref, bg_ref, bga_ref, bsl_ref,
               bf_ref, o_ref):
    cnt = jnp.sum(cntp_ref[...], axis=0)
    asum_e = jnp.sum(asump_ref[...], axis=0)
    s_sage = ssage_ref[0] + ssage_ref[1]
    s_gcn = sgcn_ref[0] + sgcn_ref[1]
    s_gat = sgat_ref[0] + sgat_ref[1]
    xwg = xwg_ref[...]
    xw = xwg[:, 0:D]
    xg = xwg[:, D:2 * D]
    a_s = xwg[:, 2 * D:2 * D + 1]
    a_d = xwg[:, 2 * D + 1:2 * D + 2]

    dinv = lax.rsqrt(cnt + 1.0)[:, None]
    h_gcn = jnp.maximum(dinv * s_gcn + dinv * dinv * xw + bg_ref[...], 0.0)

    al = a_s + a_d
    ae_self = jnp.exp(jnp.maximum(al, 0.2 * al))
    denom = asum_e[:, None] + ae_self + 1e-16
    h_gat = jnp.maximum((s_gat + ae_self * xg) / denom + bga_ref[...], 0.0)

    mean = s_sage / jnp.maximum(cnt, 1.0)[:, None]
    h_sage = jnp.maximum(
        jnp.dot(mean, wsl_ref[...], preferred_element_type=jnp.float32)
        + bsl_ref[...]
        + jnp.dot(x_ref[...], wsr_ref[...], preferred_element_type=jnp.float32),
        0.0)

    wfus = wfus_ref[...]
    o_ref[...] = (
        jnp.dot(h_gcn, wfus[0:D], preferred_element_type=jnp.float32)
        + jnp.dot(h_gat, wfus[D:2 * D], preferred_element_type=jnp.float32)
        + jnp.dot(h_sage, wfus[2 * D:3 * D], preferred_element_type=jnp.float32)
        + bf_ref[...])


def _tc_post(cnt_parts, asum_parts, s_sage, s_gcn, s_gat, x, xwg,
             W_sage_l, W_sage_r, W_fus, b_gcn, b_gat, b_sage_l, b_fus):
    return pl.pallas_call(
        _post_body,
        grid=(GRID,),
        in_specs=[
            pl.BlockSpec((NW, GB), lambda i: (0, i)),
            pl.BlockSpec((NW, GB), lambda i: (0, i)),
            pl.BlockSpec((NC, GB, D), lambda i: (0, i, 0)),
            pl.BlockSpec((NC, GB, D), lambda i: (0, i, 0)),
            pl.BlockSpec((NC, GB, D), lambda i: (0, i, 0)),
            pl.BlockSpec((GB, D), lambda i: (i, 0)),
            pl.BlockSpec((GB, 3 * D), lambda i: (i, 0)),
            pl.BlockSpec((D, D), lambda i: (0, 0)),
            pl.BlockSpec((D, D), lambda i: (0, 0)),
            pl.BlockSpec((3 * D, D), lambda i: (0, 0)),
            pl.BlockSpec((1, D), lambda i: (0, 0)),
            pl.BlockSpec((1, D), lambda i: (0, 0)),
            pl.BlockSpec((1, D), lambda i: (0, 0)),
            pl.BlockSpec((1, D), lambda i: (0, 0)),
        ],
        out_specs=pl.BlockSpec((GB, D), lambda i: (i, 0)),
        out_shape=jax.ShapeDtypeStruct((NP, D), jnp.float32),
    )(cnt_parts, asum_parts, s_sage, s_gcn, s_gat, x, xwg,
      W_sage_l, W_sage_r, W_fus,
      b_gcn.reshape(1, D), b_gat.reshape(1, D), b_sage_l.reshape(1, D),
      b_fus.reshape(1, D))


def _sc0_body(row_hbm, col_hbm, asrc_hbm, adst_hbm,
              ae_hbm, asum_hbm, cnt_hbm,
              asrc_v, adst_v, row_v, col_v, ae_v, asum_v, cnt_v):
    c = lax.axis_index("c")
    s = lax.axis_index("s")
    wid = s * NC + c
    base = wid * EPT

    pltpu.sync_copy(asrc_hbm, asrc_v)
    pltpu.sync_copy(adst_hbm, adst_v)
    pltpu.sync_copy(row_hbm.at[pl.ds(base, EPT)], row_v)
    pltpu.sync_copy(col_hbm.at[pl.ds(base, EPT)], col_v)

    zero16 = jnp.zeros((L,), jnp.float32)

    @pl.loop(0, NP, step=L)
    def _(i):
        asum_v[pl.ds(i, L)] = zero16
        cnt_v[pl.ds(i, L)] = zero16

    ones = jnp.ones((L,), jnp.float32)

    @pl.loop(0, EPT, step=L)
    def _(i):
        r = row_v[pl.ds(i, L)]
        cc = col_v[pl.ds(i, L)]
        sa = plsc.load_gather(asrc_v, [r])
        da = plsc.load_gather(adst_v, [cc])
        al = sa + da
        ae = jnp.exp(jnp.maximum(al, 0.2 * al))
        ae_v[pl.ds(i, L)] = ae
        plsc.addupdate_scatter(asum_v, [cc], ae)
        plsc.addupdate_scatter(cnt_v, [cc], ones)

    pltpu.sync_copy(ae_v, ae_hbm.at[pl.ds(base, EPT)])
    pltpu.sync_copy(asum_v, asum_hbm.at[wid])
    pltpu.sync_copy(cnt_v, cnt_hbm.at[wid])


def _sc_edge_scalars(row, col, a_src, a_dst):
    kern = pl.kernel(
        _sc0_body,
        out_type=(jax.ShapeDtypeStruct((E_PAD,), jnp.float32),
                  jax.ShapeDtypeStruct((NW, NP), jnp.float32),
                  jax.ShapeDtypeStruct((NW, NP), jnp.float32)),
        mesh=_mesh,
        scratch_types=[
            pltpu.VMEM((NP,), jnp.float32),
            pltpu.VMEM((NP,), jnp.float32),
            pltpu.VMEM((EPT,), jnp.int32),
            pltpu.VMEM((EPT,), jnp.int32),
            pltpu.VMEM((EPT,), jnp.float32),
            pltpu.VMEM((NP,), jnp.float32),
            pltpu.VMEM((NP,), jnp.float32),
        ],
        compiler_params=_sc_params,
    )
    return kern(row, col, a_src, a_dst)


NSTEP = EPT // BLK
NRING = 4


def _make_agg_body(scaled):
    def body(*refs):
        if scaled:
            (tab_hbm, row_hbm, col_hbm, ae_hbm, out_hbm,
             acc_sh, rowr, colr, aer, buf0, buf1,
             is0, is1, is2, is3, gsem0, gsem1) = refs
        else:
            (tab_hbm, row_hbm, col_hbm, out_hbm,
             acc_sh, rowr, colr, buf0, buf1,
             is0, is1, is2, is3, gsem0, gsem1) = refs
            aer = None
        c = lax.axis_index("c")
        s = lax.axis_index("s")
        wid = s * NC + c
        sbase = wid * NSTEP
        rbase = s * RPT
        isems = (is0, is1, is2, is3)
        bufs = (buf0, buf1)
        gsems = (gsem0, gsem1)

        def start_idx(step, slot):
            pltpu.async_copy(row_hbm.at[sbase + step], rowr.at[slot],
                             isems[slot])
            pltpu.async_copy(col_hbm.at[sbase + step], colr.at[slot],
                             isems[slot])
            if scaled:
                pltpu.async_copy(ae_hbm.at[sbase + step], aer.at[slot],
                                 isems[slot])

        def wait_idx(slot):
            pltpu.make_async_copy(row_hbm.at[0], rowr.at[slot],
                                  isems[slot]).wait()
            pltpu.make_async_copy(col_hbm.at[0], colr.at[slot],
                                  isems[slot]).wait()
            if scaled:
                pltpu.make_async_copy(ae_hbm.at[0], aer.at[slot],
                                      isems[slot]).wait()

        def start_gather(slot, pb):
            pltpu.async_copy(tab_hbm.at[rowr.at[slot]], bufs[pb], gsems[pb])

        def wait_gather(pb):
            pltpu.make_async_copy(tab_hbm.at[pl.ds(0, BLK)], bufs[pb],
                                  gsems[pb]).wait()

        def scale_rows(pb, slot):
            @pl.loop(0, BLK, unroll=2)
            def _(j):
                a = plsc.load_gather(aer.at[slot],
                                     [jnp.full((L,), j, jnp.int32)])
                for k in range(D // L):
                    sl = pl.ds(k * L, L)
                    bufs[pb][j, sl] = bufs[pb][j, sl] * a

        zero16 = jnp.zeros((L,), jnp.float32)

        @pl.loop(0, BLK)
        def _(j):
            for k in range(D // L):
                buf0[j, pl.ds(k * L, L)] = zero16

        @pl.loop(0, RPT, step=BLK)
        def _(r0):
            pltpu.sync_copy(buf0, acc_sh.at[pl.ds(rbase + r0, BLK)])

        for slot in range(NRING):
            start_idx(slot, slot)
        for pb in range(2):
            wait_idx(pb)
            start_gather(pb, pb)

        plsc.subcore_barrier()

        @pl.loop(0, NSTEP, step=NRING)
        def _(g):
            for b in range(NRING):
                gi = g + b
                pb = b % 2
                wait_gather(pb)
                if scaled:
                    scale_rows(pb, b)
                pltpu.sync_copy(bufs[pb], acc_sh.at[colr.at[b]], add=True)

                @pl.when(gi + NRING < NSTEP)
                def _():
                    start_idx(gi + NRING, b)

                @pl.when(gi + 2 < NSTEP)
                def _():
                    wait_idx((b + 2) % NRING)
                    start_gather((b + 2) % NRING, pb)

        plsc.subcore_barrier()

        @pl.loop(0, RPT, step=BLK)
        def _(r0):
            pltpu.sync_copy(acc_sh.at[pl.ds(rbase + r0, BLK)], buf0)
            pltpu.sync_copy(buf0, out_hbm.at[c].at[pl.ds(rbase + r0, BLK)])

    return body


def _make_agg(scaled):
    scratch = [
        pltpu.VMEM_SHARED((NP, D), jnp.float32),
        pltpu.VMEM((NRING, BLK), jnp.int32),
        pltpu.VMEM((NRING, BLK), jnp.int32),
    ]
    if scaled:
        scratch.append(pltpu.VMEM((NRING, BLK), jnp.float32))
    scratch += [
        pltpu.VMEM((BLK, D), jnp.float32),
        pltpu.VMEM((BLK, D), jnp.float32),
    ]
    scratch += [pltpu.SemaphoreType.DMA] * (NRING + 2)
    return pl.kernel(
        _make_agg_body(scaled),
        out_type=jax.ShapeDtypeStruct((NC, NP, D), jnp.float32),
        mesh=_mesh,
        scratch_types=scratch,
        compiler_params=_sc_params,
    )


_sc_agg_plain = _make_agg(False)
_sc_agg_scaled = _make_agg(True)


def kernel(x, edge_index, W_gcn, b_gcn, W_gat, att_src, att_dst, b_gat,
           W_sage_l, b_sage_l, W_sage_r, W_fus, b_fus):
    row = edge_index[0]
    col = edge_index[1]
    npad = E_PAD - row.shape[0]
    ar = jnp.arange(npad, dtype=jnp.int32)
    row_p = jnp.concatenate([row, (ar * 37) % N])
    col_p = jnp.concatenate([col, N + ar % (NP - N)])

    x_p = jnp.zeros((NP, D), jnp.float32).at[:N].set(x)
    att2 = (jnp.zeros((D, D), jnp.float32)
            .at[:, 0].set(W_gat @ att_src).at[:, 1].set(W_gat @ att_dst))
    wcat = jnp.concatenate([W_gcn, W_gat, att2], axis=1)

    xwg = _tc_pre(x_p, wcat)
    xw = xwg[:, 0:D]
    xg = xwg[:, D:2 * D]
    a_src = xwg[:, 2 * D]
    a_dst = xwg[:, 2 * D + 1]

    row2 = row_p.reshape(NW * NSTEP, BLK)
    col2 = col_p.reshape(NW * NSTEP, BLK)
    ae, asum_parts, cnt_parts = _sc_edge_scalars(row_p, col_p, a_src, a_dst)
    ae2 = ae.reshape(NW * NSTEP, BLK)
    s_sage = _sc_agg_plain(x_p, row2, col2)
    u = _tc_mid(cnt_parts, xw)
    s_gcn = _sc_agg_plain(u, row2, col2)
    s_gat = _sc_agg_scaled(xg, row2, col2, ae2)

    out = _tc_post(cnt_parts, asum_parts, s_sage, s_gcn, s_gat, x_p, xwg,
                   W_sage_l, W_sage_r, W_fus, b_gcn, b_gat, b_sage_l, b_fus)
    return out[:N]


# speedup vs baseline: 43.0430x; 1.1004x over previous
import dataclasses

import jax
import jax.numpy as jnp
from jax import lax
from jax.experimental import pallas as pl
from jax.experimental.pallas import tpu as pltpu
from jax.experimental.pallas import tpu_sc as plsc

N = 10000
NP = 10240
D = 128
NC = 2
NS = 16
L = 16
NW = NC * NS
EPT = 10240
E_PAD = NW * EPT
BLK = 128
RPT = NP // NS
GB = 1024
GRID = NP // GB

_mesh = plsc.VectorSubcoreMesh(core_axis_name="c", subcore_axis_name="s")

_sc_params = pltpu.CompilerParams()
if "needs_layout_passes" in pltpu.CompilerParams.__dataclass_fields__:
    _sc_params = dataclasses.replace(_sc_params, needs_layout_passes=False)


def _pre_body(x_ref, w_ref, o_ref):
    o_ref[...] = jnp.dot(x_ref[...], w_ref[...],
                         preferred_element_type=jnp.float32)


def _tc_pre(x, wcat):
    return pl.pallas_call(
        _pre_body,
        grid=(GRID,),
        in_specs=[pl.BlockSpec((GB, D), lambda i: (i, 0)),
                  pl.BlockSpec((D, 3 * D), lambda i: (0, 0))],
        out_specs=pl.BlockSpec((GB, 3 * D), lambda i: (i, 0)),
        out_shape=jax.ShapeDtypeStruct((NP, 3 * D), jnp.float32),
    )(x, wcat)


def _mid_body(cntp_ref, xw_ref, u_ref):
    cnt = jnp.sum(cntp_ref[...], axis=0)
    dinv = lax.rsqrt(cnt + 1.0)
    u_ref[...] = dinv[:, None] * xw_ref[...]


def _tc_mid(cnt_parts, xw):
    return pl.pallas_call(
        _mid_body,
        grid=(GRID,),
        in_specs=[pl.BlockSpec((NW, GB), lambda i: (0, i)),
                  pl.BlockSpec((GB, D), lambda i: (i, 0))],
        out_specs=pl.BlockSpec((GB, D), lambda i: (i, 0)),
        out_shape=jax.ShapeDtypeStruct((NP, D), jnp.float32),
    )(cnt_parts, xw)


def _post_body(cntp_ref, asump_ref, ssage_ref, sgcn_ref, sgat_ref, x_ref,
               xwg_ref, wsl_ref, wsr_ref, wfus_ref, bg_ref, bga_ref, bsl_ref,
               bf_ref, o_ref):
    cnt = jnp.sum(cntp_ref[...], axis=0)
    asum_e = jnp.sum(asump_ref[...], axis=0)
    s_sage = ssage_ref[0] + ssage_ref[1]
    s_gcn = sgcn_ref[0] + sgcn_ref[1]
    s_gat = sgat_ref[0] + sgat_ref[1]
    xwg = xwg_ref[...]
    xw = xwg[:, 0:D]
    xg = xwg[:, D:2 * D]
    a_s = xwg[:, 2 * D:2 * D + 1]
    a_d = xwg[:, 2 * D + 1:2 * D + 2]

    dinv = lax.rsqrt(cnt + 1.0)[:, None]
    h_gcn = jnp.maximum(dinv * s_gcn + dinv * dinv * xw + bg_ref[...], 0.0)

    al = a_s + a_d
    ae_self = jnp.exp(jnp.maximum(al, 0.2 * al))
    denom = asum_e[:, None] + ae_self + 1e-16
    h_gat = jnp.maximum((s_gat + ae_self * xg) / denom + bga_ref[...], 0.0)

    mean = s_sage / jnp.maximum(cnt, 1.0)[:, None]
    h_sage = jnp.maximum(
        jnp.dot(mean, wsl_ref[...], preferred_element_type=jnp.float32)
        + bsl_ref[...]
        + jnp.dot(x_ref[...], wsr_ref[...], preferred_element_type=jnp.float32),
        0.0)

    wfus = wfus_ref[...]
    o_ref[...] = (
        jnp.dot(h_gcn, wfus[0:D], preferred_element_type=jnp.float32)
        + jnp.dot(h_gat, wfus[D:2 * D], preferred_element_type=jnp.float32)
        + jnp.dot(h_sage, wfus[2 * D:3 * D], preferred_element_type=jnp.float32)
        + bf_ref[...])


def _tc_post(cnt_parts, asum_parts, s_sage, s_gcn, s_gat, x, xwg,
             W_sage_l, W_sage_r, W_fus, b_gcn, b_gat, b_sage_l, b_fus):
    return pl.pallas_call(
        _post_body,
        grid=(GRID,),
        in_specs=[
            pl.BlockSpec((NW, GB), lambda i: (0, i)),
            pl.BlockSpec((NW, GB), lambda i: (0, i)),
            pl.BlockSpec((NC, GB, D), lambda i: (0, i, 0)),
            pl.BlockSpec((NC, GB, D), lambda i: (0, i, 0)),
            pl.BlockSpec((NC, GB, D), lambda i: (0, i, 0)),
            pl.BlockSpec((GB, D), lambda i: (i, 0)),
            pl.BlockSpec((GB, 3 * D), lambda i: (i, 0)),
            pl.BlockSpec((D, D), lambda i: (0, 0)),
            pl.BlockSpec((D, D), lambda i: (0, 0)),
            pl.BlockSpec((3 * D, D), lambda i: (0, 0)),
            pl.BlockSpec((1, D), lambda i: (0, 0)),
            pl.BlockSpec((1, D), lambda i: (0, 0)),
            pl.BlockSpec((1, D), lambda i: (0, 0)),
            pl.BlockSpec((1, D), lambda i: (0, 0)),
        ],
        out_specs=pl.BlockSpec((GB, D), lambda i: (i, 0)),
        out_shape=jax.ShapeDtypeStruct((NP, D), jnp.float32),
    )(cnt_parts, asum_parts, s_sage, s_gcn, s_gat, x, xwg,
      W_sage_l, W_sage_r, W_fus,
      b_gcn.reshape(1, D), b_gat.reshape(1, D), b_sage_l.reshape(1, D),
      b_fus.reshape(1, D))


def _sc0_body(row_hbm, col_hbm, asrc_hbm, adst_hbm,
              ae_hbm, asum_hbm, cnt_hbm,
              asrc_v, adst_v, row_v, col_v, ae_v, asum_v, cnt_v):
    c = lax.axis_index("c")
    s = lax.axis_index("s")
    wid = s * NC + c
    base = wid * EPT

    pltpu.sync_copy(asrc_hbm, asrc_v)
    pltpu.sync_copy(adst_hbm, adst_v)
    pltpu.sync_copy(row_hbm.at[pl.ds(base, EPT)], row_v)
    pltpu.sync_copy(col_hbm.at[pl.ds(base, EPT)], col_v)

    zero16 = jnp.zeros((L,), jnp.float32)

    @pl.loop(0, NP, step=L)
    def _(i):
        asum_v[pl.ds(i, L)] = zero16
        cnt_v[pl.ds(i, L)] = zero16

    ones = jnp.ones((L,), jnp.float32)

    @pl.loop(0, EPT, step=L)
    def _(i):
        r = row_v[pl.ds(i, L)]
        cc = col_v[pl.ds(i, L)]
        sa = plsc.load_gather(asrc_v, [r])
        da = plsc.load_gather(adst_v, [cc])
        al = sa + da
        ae = jnp.exp(jnp.maximum(al, 0.2 * al))
        ae_v[pl.ds(i, L)] = ae
        plsc.addupdate_scatter(asum_v, [cc], ae)
        plsc.addupdate_scatter(cnt_v, [cc], ones)

    pltpu.sync_copy(ae_v, ae_hbm.at[pl.ds(base, EPT)])
    pltpu.sync_copy(asum_v, asum_hbm.at[wid])
    pltpu.sync_copy(cnt_v, cnt_hbm.at[wid])


def _sc_edge_scalars(row, col, a_src, a_dst):
    kern = pl.kernel(
        _sc0_body,
        out_type=(jax.ShapeDtypeStruct((E_PAD,), jnp.float32),
                  jax.ShapeDtypeStruct((NW, NP), jnp.float32),
                  jax.ShapeDtypeStruct((NW, NP), jnp.float32)),
        mesh=_mesh,
        scratch_types=[
            pltpu.VMEM((NP,), jnp.float32),
            pltpu.VMEM((NP,), jnp.float32),
            pltpu.VMEM((EPT,), jnp.int32),
            pltpu.VMEM((EPT,), jnp.int32),
            pltpu.VMEM((EPT,), jnp.float32),
            pltpu.VMEM((NP,), jnp.float32),
            pltpu.VMEM((NP,), jnp.float32),
        ],
        compiler_params=_sc_params,
    )
    return kern(row, col, a_src, a_dst)


FBLK = 64
FNSTEP = EPT // FBLK
NBUF = 4
NRING = 8
DRB = 64


def _make_agg_body(scaled):
    def body(*refs):
        if scaled:
            (tab_hbm, row_hbm, col_hbm, ae_hbm, out_hbm, acc_sh,
             rowr, colr, aer, b0, b1, b2, b3, zb,
             is0, is1, is2, is3, is4, is5, is6, is7,
             gs0, gs1, gs2, gs3, ss0, ss1, ss2, ss3) = refs
        else:
            (tab_hbm, row_hbm, col_hbm, out_hbm, acc_sh,
             rowr, colr, b0, b1, b2, b3, zb,
             is0, is1, is2, is3, is4, is5, is6, is7,
             gs0, gs1, gs2, gs3, ss0, ss1, ss2, ss3) = refs
            aer = None
        c = lax.axis_index("c")
        s = lax.axis_index("s")
        wid = s * NC + c
        sbase = wid * FNSTEP
        rbase = s * RPT
        isems = (is0, is1, is2, is3, is4, is5, is6, is7)
        bufs = (b0, b1, b2, b3)
        gsems = (gs0, gs1, gs2, gs3)
        ssems = (ss0, ss1, ss2, ss3)

        def start_idx(step, slot):
            pltpu.async_copy(row_hbm.at[sbase + step], rowr.at[slot],
                             isems[slot])
            pltpu.async_copy(col_hbm.at[sbase + step], colr.at[slot],
                             isems[slot])
            if scaled:
                pltpu.async_copy(ae_hbm.at[sbase + step], aer.at[slot],
                                 isems[slot])

        def wait_idx(slot):
            pltpu.make_async_copy(row_hbm.at[0], rowr.at[slot],
                                  isems[slot]).wait()
            pltpu.make_async_copy(col_hbm.at[0], colr.at[slot],
                                  isems[slot]).wait()
            if scaled:
                pltpu.make_async_copy(ae_hbm.at[0], aer.at[slot],
                                      isems[slot]).wait()

        def start_gather(slot, pb):
            pltpu.async_copy(tab_hbm.at[rowr.at[slot]], bufs[pb], gsems[pb])

        def wait_gather(pb):
            pltpu.make_async_copy(tab_hbm.at[pl.ds(0, FBLK)], bufs[pb],
                                  gsems[pb]).wait()

        def wait_scatter(pb):
            pltpu.make_async_copy(tab_hbm.at[pl.ds(0, FBLK)], bufs[pb],
                                  ssems[pb]).wait()

        def scale_rows(pb, slot):
            @pl.loop(0, FBLK, unroll=2)
            def _(j):
                a = plsc.load_gather(aer.at[slot],
                                     [jnp.full((L,), j, jnp.int32)])
                for k in range(D // L):
                    sl = pl.ds(k * L, L)
                    bufs[pb][j, sl] = bufs[pb][j, sl] * a

        zero16 = jnp.zeros((L,), jnp.float32)

        @pl.loop(0, DRB)
        def _(j):
            for k in range(D // L):
                zb[j, pl.ds(k * L, L)] = zero16

        @pl.loop(0, RPT, step=DRB)
        def _(r0):
            pltpu.sync_copy(zb, acc_sh.at[pl.ds(rbase + r0, DRB)])

        for k in range(NBUF):
            start_idx(k, k)
        for k in range(NBUF - 1):
            wait_idx(k)
            start_gather(k, k)

        plsc.subcore_barrier()

        @pl.loop(0, FNSTEP, step=NRING)
        def _(g):
            for b in range(NRING):
                gi = g + b
                s8 = b
                b4 = b % NBUF
                wait_gather(b4)
                if scaled:
                    scale_rows(b4, s8)
                pltpu.async_copy(bufs[b4], acc_sh.at[colr.at[s8]],
                                 ssems[b4], add=True)

                @pl.when(gi + NBUF < FNSTEP)
                def _():
                    start_idx(gi + NBUF, (b + NBUF) % NRING)

                @pl.when(gi + 3 < FNSTEP)
                def _():
                    @pl.when(gi >= 1)
                    def _():
                        wait_scatter((b + 3) % NBUF)

                    wait_idx((b + 3) % NRING)
                    start_gather((b + 3) % NRING, (b + 3) % NBUF)

        for k in range(NBUF):
            wait_scatter(k)
        plsc.subcore_barrier()

        @pl.loop(0, RPT, step=DRB)
        def _(r0):
            pltpu.sync_copy(acc_sh.at[pl.ds(rbase + r0, DRB)], zb)
            pltpu.sync_copy(zb, out_hbm.at[c].at[pl.ds(rbase + r0, DRB)])

    return body


def _make_agg(scaled):
    scratch = [
        pltpu.VMEM_SHARED((NP, D), jnp.float32),
        pltpu.VMEM((NRING, FBLK), jnp.int32),
        pltpu.VMEM((NRING, FBLK), jnp.int32),
    ]
    if scaled:
        scratch.append(pltpu.VMEM((NRING, FBLK), jnp.float32))
    scratch += [pltpu.VMEM((FBLK, D), jnp.float32)] * NBUF
    scratch += [pltpu.VMEM((DRB, D), jnp.float32)]
    scratch += [pltpu.SemaphoreType.DMA] * (NRING + 2 * NBUF)
    return pl.kernel(
        _make_agg_body(scaled),
        out_type=jax.ShapeDtypeStruct((NC, NP, D), jnp.float32),
        mesh=_mesh,
        scratch_types=scratch,
        compiler_params=_sc_params,
    )


_sc_agg_plain = _make_agg(False)
_sc_agg_scaled = _make_agg(True)


def kernel(x, edge_index, W_gcn, b_gcn, W_gat, att_src, att_dst, b_gat,
           W_sage_l, b_sage_l, W_sage_r, W_fus, b_fus):
    row = edge_index[0]
    col = edge_index[1]
    npad = E_PAD - row.shape[0]
    ar = jnp.arange(npad, dtype=jnp.int32)
    row_p = jnp.concatenate([row, (ar * 37) % N])
    col_p = jnp.concatenate([col, N + ar % (NP - N)])

    x_p = jnp.zeros((NP, D), jnp.float32).at[:N].set(x)
    att2 = (jnp.zeros((D, D), jnp.float32)
            .at[:, 0].set(W_gat @ att_src).at[:, 1].set(W_gat @ att_dst))
    wcat = jnp.concatenate([W_gcn, W_gat, att2], axis=1)

    xwg = _tc_pre(x_p, wcat)
    xw = xwg[:, 0:D]
    xg = xwg[:, D:2 * D]
    a_src = xwg[:, 2 * D]
    a_dst = xwg[:, 2 * D + 1]

    row2 = row_p.reshape(NW * FNSTEP, FBLK)
    col2 = col_p.reshape(NW * FNSTEP, FBLK)
    ae, asum_parts, cnt_parts = _sc_edge_scalars(row_p, col_p, a_src, a_dst)
    ae2 = ae.reshape(NW * FNSTEP, FBLK)
    s_sage = _sc_agg_plain(x_p, row2, col2)
    u = _tc_mid(cnt_parts, xw)
    s_gcn = _sc_agg_plain(u, row2, col2)
    s_gat = _sc_agg_scaled(xg, row2, col2, ae2)

    out = _tc_post(cnt_parts, asum_parts, s_sage, s_gcn, s_gat, x_p, xwg,
                   W_sage_l, W_sage_r, W_fus, b_gcn, b_gat, b_sage_l, b_fus)
    return out[:N]
